# baseline (speedup 1.0000x reference)
import jax
import jax.numpy as jnp
from jax import lax
from jax.experimental import pallas as pl
from jax.experimental.pallas import tpu as pltpu
from jax.experimental.pallas import tpu_sc as plsc

N_NODES = 10000
N_USERS = 4000
N_ITEMS = 6000
D_FEAT = 128
N_EDGES = 320000
CONVS = 3

L = 16
F_PER_TILE = 4
C = 1280
NCH = N_EDGES // C
E_PAD = NCH * C


TILE_W = F_PER_TILE * N_NODES


def _sc_body(emb_hbm, sd_hbm, w_hbm, out_hbm,
             tab_a, tab_b, tot, sdbuf, wbuf, sems):
    c = lax.axis_index("c")
    s = lax.axis_index("s")
    tile = c * 16 + s
    rows = pl.ds(tile * TILE_W, TILE_W)

    def issue(b, ch):
        off = pl.ds(ch * C, C)
        pltpu.async_copy(sd_hbm.at[off], sdbuf.at[b], sems.at[b])
        pltpu.async_copy(w_hbm.at[off], wbuf.at[b], sems.at[b])

    def drain(b, ch):
        off = pl.ds(ch * C, C)
        pltpu.make_async_copy(sd_hbm.at[off], sdbuf.at[b], sems.at[b]).wait()
        pltpu.make_async_copy(w_hbm.at[off], wbuf.at[b], sems.at[b]).wait()

    def zero(ref):
        z = jnp.zeros((L,), jnp.float32)

        @plsc.parallel_loop(0, TILE_W // L, unroll=4)
        def _(i):
            ref[pl.ds(i * L, L)] = z

    def acc_tot(ref):
        @plsc.parallel_loop(0, TILE_W // L, unroll=4)
        def _(i):
            sl = pl.ds(i * L, L)
            tot[sl] = tot[sl] + ref[sl]

    def process(b, rd, wr):
        rds = [rd.at[pl.ds(f * N_NODES, N_NODES)] for f in range(F_PER_TILE)]
        wrs = [wr.at[pl.ds(f * N_NODES, N_NODES)] for f in range(F_PER_TILE)]

        @plsc.parallel_loop(0, C // L, unroll=4)
        def _(i):
            sl = pl.ds(i * L, L)
            sd16 = sdbuf[b, sl]
            s16 = jax.lax.shift_right_logical(sd16, 14)
            d16 = jax.lax.bitwise_and(sd16, jnp.int32(16383))
            w16 = wbuf[b, sl]
            gs = [plsc.load_gather(rds[f], [s16]) for f in range(F_PER_TILE)]
            for f in range(F_PER_TILE):
                plsc.addupdate_scatter(wrs[f], [d16], gs[f] * w16)

    def edge_pass(rd, wr):
        @pl.loop(0, NCH, step=2)
        def _(g):
            for b in range(2):
                ch = g + b
                drain(b, ch)
                process(b, rd, wr)
                nxt = ch + 2

                @pl.when(nxt < NCH)
                def _():
                    issue(b, nxt)

    def prime():
        issue(0, 0)
        issue(1, 1)

    da = pltpu.async_copy(emb_hbm.at[rows], tab_a, sems.at[2])
    dt = pltpu.async_copy(emb_hbm.at[rows], tot, sems.at[2])
    prime()
    zero(tab_b)
    da.wait()
    dt.wait()

    edge_pass(tab_a, tab_b)
    prime()
    acc_tot(tab_b)
    zero(tab_a)
    edge_pass(tab_b, tab_a)
    prime()
    acc_tot(tab_a)
    zero(tab_b)
    edge_pass(tab_a, tab_b)

    @pl.loop(0, TILE_W // L, unroll=4)
    def _(i):
        sl = pl.ds(i * L, L)
        tab_a[sl] = (tot[sl] + tab_b[sl]) * 0.25
    pltpu.async_copy(tab_a, out_hbm.at[rows], sems.at[2]).wait()


def kernel(all_emb, edge_index, edge_weight):
    sd = (edge_index[0].astype(jnp.int32) << 14) | edge_index[1].astype(jnp.int32)
    w = edge_weight.astype(jnp.float32)
    emb_t = all_emb.T.reshape(-1)

    sc_kernel = pl.kernel(
        _sc_body,
        out_type=jax.ShapeDtypeStruct((D_FEAT * N_NODES,), jnp.float32),
        mesh=plsc.VectorSubcoreMesh(core_axis_name="c", subcore_axis_name="s"),
        compiler_params=pltpu.CompilerParams(needs_layout_passes=False),
        scratch_types=[
            pltpu.VMEM((TILE_W,), jnp.float32),
            pltpu.VMEM((TILE_W,), jnp.float32),
            pltpu.VMEM((TILE_W,), jnp.float32),
            pltpu.VMEM((2, C), jnp.int32),
            pltpu.VMEM((2, C), jnp.float32),
            pltpu.SemaphoreType.DMA((3,)),
        ],
    )
    out_t = sc_kernel(emb_t, sd, w)
    light_out = out_t.reshape(D_FEAT, N_NODES).T
    return (light_out[:N_USERS], light_out[N_USERS:N_USERS + N_ITEMS])


# speedup vs baseline: 1.0139x; 1.0139x over previous
import jax
import jax.numpy as jnp
from jax import lax
from jax.experimental import pallas as pl
from jax.experimental.pallas import tpu as pltpu
from jax.experimental.pallas import tpu_sc as plsc

N_NODES = 10000
N_USERS = 4000
N_ITEMS = 6000
D_FEAT = 128
N_EDGES = 320000
CONVS = 3

L = 16
F_PER_TILE = 4
C = 1280
NCH = N_EDGES // C
E_PAD = NCH * C


TILE_W = F_PER_TILE * N_NODES


def _sc_body(emb_hbm, sd_hbm, w_hbm, out_hbm,
             tab_a, tab_b, tot, sdbuf, wbuf, sems):
    c = lax.axis_index("c")
    s = lax.axis_index("s")
    tile = c * 16 + s
    rows = pl.ds(tile * TILE_W, TILE_W)

    def issue(b, ch):
        off = pl.ds(ch * C, C)
        pltpu.async_copy(sd_hbm.at[off], sdbuf.at[b], sems.at[b])
        pltpu.async_copy(w_hbm.at[off], wbuf.at[b], sems.at[b])

    def drain(b, ch):
        off = pl.ds(ch * C, C)
        pltpu.make_async_copy(sd_hbm.at[off], sdbuf.at[b], sems.at[b]).wait()
        pltpu.make_async_copy(w_hbm.at[off], wbuf.at[b], sems.at[b]).wait()

    def zero(ref):
        z = jnp.zeros((L,), jnp.float32)

        @plsc.parallel_loop(0, TILE_W // L, unroll=4)
        def _(i):
            ref[pl.ds(i * L, L)] = z

    def acc_tot(ref):
        @plsc.parallel_loop(0, TILE_W // L, unroll=4)
        def _(i):
            sl = pl.ds(i * L, L)
            tot[sl] = tot[sl] + ref[sl]

    def process(b, rd, wr):
        rds = [rd.at[pl.ds(f * N_NODES, N_NODES)] for f in range(F_PER_TILE)]
        wrs = [wr.at[pl.ds(f * N_NODES, N_NODES)] for f in range(F_PER_TILE)]

        @plsc.parallel_loop(0, C // L, unroll=4)
        def _(i):
            sl = pl.ds(i * L, L)
            sd16 = sdbuf[b, sl]
            s16 = jax.lax.shift_right_logical(sd16, 14)
            d16 = jax.lax.bitwise_and(sd16, jnp.int32(16383))
            w16 = wbuf[b, sl]
            for f in range(F_PER_TILE):
                g = plsc.load_gather(rds[f], [s16])
                plsc.addupdate_scatter(wrs[f], [d16], g * w16)

    def edge_pass(rd, wr):
        @pl.loop(0, NCH, step=2)
        def _(g):
            for b in range(2):
                ch = g + b
                drain(b, ch)
                process(b, rd, wr)
                nxt = ch + 2

                @pl.when(nxt < NCH)
                def _():
                    issue(b, nxt)

    def prime():
        issue(0, 0)
        issue(1, 1)

    da = pltpu.async_copy(emb_hbm.at[rows], tab_a, sems.at[2])
    dt = pltpu.async_copy(emb_hbm.at[rows], tot, sems.at[2])
    prime()
    zero(tab_b)
    da.wait()
    dt.wait()

    edge_pass(tab_a, tab_b)
    prime()
    acc_tot(tab_b)
    zero(tab_a)
    edge_pass(tab_b, tab_a)
    prime()
    acc_tot(tab_a)
    zero(tab_b)
    edge_pass(tab_a, tab_b)

    @pl.loop(0, TILE_W // L, unroll=4)
    def _(i):
        sl = pl.ds(i * L, L)
        tab_a[sl] = (tot[sl] + tab_b[sl]) * 0.25
    pltpu.async_copy(tab_a, out_hbm.at[rows], sems.at[2]).wait()


def kernel(all_emb, edge_index, edge_weight):
    sd = (edge_index[0].astype(jnp.int32) << 14) | edge_index[1].astype(jnp.int32)
    w = edge_weight.astype(jnp.float32)
    emb_t = all_emb.T.reshape(-1)

    sc_kernel = pl.kernel(
        _sc_body,
        out_type=jax.ShapeDtypeStruct((D_FEAT * N_NODES,), jnp.float32),
        mesh=plsc.VectorSubcoreMesh(core_axis_name="c", subcore_axis_name="s"),
        compiler_params=pltpu.CompilerParams(needs_layout_passes=False),
        scratch_types=[
            pltpu.VMEM((TILE_W,), jnp.float32),
            pltpu.VMEM((TILE_W,), jnp.float32),
            pltpu.VMEM((TILE_W,), jnp.float32),
            pltpu.VMEM((2, C), jnp.int32),
            pltpu.VMEM((2, C), jnp.float32),
            pltpu.SemaphoreType.DMA((3,)),
        ],
    )
    out_t = sc_kernel(emb_t, sd, w)
    light_out = out_t.reshape(D_FEAT, N_NODES).T
    return (light_out[:N_USERS], light_out[N_USERS:N_USERS + N_ITEMS])


# speedup vs baseline: 1.0148x; 1.0009x over previous
import jax
import jax.numpy as jnp
from jax import lax
from jax.experimental import pallas as pl
from jax.experimental.pallas import tpu as pltpu
from jax.experimental.pallas import tpu_sc as plsc

N_NODES = 10000
N_USERS = 4000
N_ITEMS = 6000
D_FEAT = 128
N_EDGES = 320000
N_CONVS = 3

L = 16
F_PER_TILE = 4
C = 1280
NCH = N_EDGES // C


TILE_W = F_PER_TILE * N_NODES


def _sc_body(emb_hbm, sd_hbm, w_hbm, out_hbm,
             tab_a, tab_b, tot, sdbuf, wbuf, sems):
    c = lax.axis_index("c")
    s = lax.axis_index("s")
    tile = c * 16 + s
    rows = pl.ds(tile * TILE_W, TILE_W)

    def issue(b, ch):
        off = pl.ds(ch * C, C)
        pltpu.async_copy(sd_hbm.at[off], sdbuf.at[b], sems.at[b])
        pltpu.async_copy(w_hbm.at[off], wbuf.at[b], sems.at[b])

    def drain(b, ch):
        off = pl.ds(ch * C, C)
        pltpu.make_async_copy(sd_hbm.at[off], sdbuf.at[b], sems.at[b]).wait()
        pltpu.make_async_copy(w_hbm.at[off], wbuf.at[b], sems.at[b]).wait()

    def zero(ref):
        z = jnp.zeros((L,), jnp.float32)

        @plsc.parallel_loop(0, TILE_W // L, unroll=4)
        def _(i):
            ref[pl.ds(i * L, L)] = z

    def acc_tot(ref):
        @plsc.parallel_loop(0, TILE_W // L, unroll=4)
        def _(i):
            sl = pl.ds(i * L, L)
            tot[sl] = tot[sl] + ref[sl]

    def process(b, rd, wr):
        rds = [rd.at[pl.ds(f * N_NODES, N_NODES)] for f in range(F_PER_TILE)]
        wrs = [wr.at[pl.ds(f * N_NODES, N_NODES)] for f in range(F_PER_TILE)]

        @plsc.parallel_loop(0, C // L, unroll=4)
        def _(i):
            sl = pl.ds(i * L, L)
            sd16 = sdbuf[b, sl]
            s16 = jax.lax.shift_right_logical(sd16, 14)
            d16 = jax.lax.bitwise_and(sd16, jnp.int32(16383))
            w16 = wbuf[b, sl]
            for f in range(F_PER_TILE):
                g = plsc.load_gather(rds[f], [s16])
                plsc.addupdate_scatter(wrs[f], [d16], g * w16)

    def edge_pass(rd, wr):
        @pl.loop(0, NCH, step=2)
        def _(g):
            for b in range(2):
                ch = g + b
                drain(b, ch)
                process(b, rd, wr)
                nxt = ch + 2

                @pl.when(nxt < NCH)
                def _():
                    issue(b, nxt)

    def prime():
        issue(0, 0)
        issue(1, 1)

    da = pltpu.async_copy(emb_hbm.at[rows], tab_a, sems.at[2])
    dt = pltpu.async_copy(emb_hbm.at[rows], tot, sems.at[2])
    prime()
    zero(tab_b)
    da.wait()
    dt.wait()

    edge_pass(tab_a, tab_b)
    prime()
    acc_tot(tab_b)
    zero(tab_a)
    edge_pass(tab_b, tab_a)
    prime()
    acc_tot(tab_a)
    zero(tab_b)
    edge_pass(tab_a, tab_b)

    @pl.loop(0, TILE_W // L, unroll=4)
    def _(i):
        sl = pl.ds(i * L, L)
        tab_a[sl] = (tot[sl] + tab_b[sl]) * 0.25
    pltpu.async_copy(tab_a, out_hbm.at[rows], sems.at[2]).wait()


def kernel(all_emb, edge_index, edge_weight):
    sd = (edge_index[0].astype(jnp.int32) << 14) | edge_index[1].astype(jnp.int32)
    w = edge_weight.astype(jnp.float32)
    emb_t = all_emb.T.reshape(-1)

    sc_kernel = pl.kernel(
        _sc_body,
        out_type=jax.ShapeDtypeStruct((D_FEAT * N_NODES,), jnp.float32),
        mesh=plsc.VectorSubcoreMesh(core_axis_name="c", subcore_axis_name="s"),
        compiler_params=pltpu.CompilerParams(needs_layout_passes=False),
        scratch_types=[
            pltpu.VMEM((TILE_W,), jnp.float32),
            pltpu.VMEM((TILE_W,), jnp.float32),
            pltpu.VMEM((TILE_W,), jnp.float32),
            pltpu.VMEM((2, C), jnp.int32),
            pltpu.VMEM((2, C), jnp.float32),
            pltpu.SemaphoreType.DMA((3,)),
        ],
    )
    out_t = sc_kernel(emb_t, sd, w)
    light_out = out_t.reshape(D_FEAT, N_NODES).T
    return (light_out[:N_USERS], light_out[N_USERS:N_USERS + N_ITEMS])
